# baseline (speedup 1.0000x reference)
import functools

import jax
import jax.numpy as jnp
from jax import lax
from jax.experimental import pallas as pl
from jax.experimental.pallas import tpu as pltpu
from jax.experimental.pallas import tpu_sc as plsc

EMB_DIM = 64
BATCH = 16384
NUM_WORKERS = 32
ROWS_PER_WORKER = BATCH // NUM_WORKERS
MLP_BLOCK = 2048


def _sc_gather_body(user_table, user_id, book_table, book_id,
                    ue_out, be_out,
                    uidx_v, urows_v, bidx_v, brows_v, usem, bsem):
    wid = lax.axis_index("s") * 2 + lax.axis_index("c")
    base = wid * ROWS_PER_WORKER
    pltpu.sync_copy(user_id.at[pl.ds(base, ROWS_PER_WORKER)], uidx_v)
    ucopy = pltpu.async_copy(user_table.at[uidx_v], urows_v, usem)
    pltpu.sync_copy(book_id.at[pl.ds(base, ROWS_PER_WORKER)], bidx_v)
    bcopy = pltpu.async_copy(book_table.at[bidx_v], brows_v, bsem)
    ucopy.wait()
    pltpu.sync_copy(urows_v, ue_out.at[pl.ds(base, ROWS_PER_WORKER)])
    bcopy.wait()
    pltpu.sync_copy(brows_v, be_out.at[pl.ds(base, ROWS_PER_WORKER)])


@functools.lru_cache(maxsize=1)
def _sc_gather():
    return pl.kernel(
        _sc_gather_body,
        out_type=[
            jax.ShapeDtypeStruct((BATCH, EMB_DIM), jnp.float32),
            jax.ShapeDtypeStruct((BATCH, EMB_DIM), jnp.float32),
        ],
        mesh=plsc.VectorSubcoreMesh(core_axis_name="c", subcore_axis_name="s"),
        scratch_types=[
            pltpu.VMEM((ROWS_PER_WORKER,), jnp.int32),
            pltpu.VMEM((ROWS_PER_WORKER, EMB_DIM), jnp.float32),
            pltpu.VMEM((ROWS_PER_WORKER,), jnp.int32),
            pltpu.VMEM((ROWS_PER_WORKER, EMB_DIM), jnp.float32),
            pltpu.SemaphoreType.DMA,
            pltpu.SemaphoreType.DMA,
        ],
        compiler_params=pltpu.CompilerParams(use_tc_tiling_on_sc=False),
    )


def _mlp_body(ue_ref, be_ref, w1u_ref, w1b_ref, b1_ref, w2_ref, b2_ref,
              w3_ref, b3_ref, out_ref):
    h = jnp.dot(ue_ref[...], w1u_ref[...], preferred_element_type=jnp.float32)
    h += jnp.dot(be_ref[...], w1b_ref[...], preferred_element_type=jnp.float32)
    h = jnp.maximum(h + b1_ref[...], 0.0)
    h = jnp.dot(h, w2_ref[...], preferred_element_type=jnp.float32)
    h = jnp.maximum(h + b2_ref[...], 0.0)
    out_ref[...] = (
        jnp.sum(h * w3_ref[...], axis=1, keepdims=True) + b3_ref[...]
    )


def kernel(user_id, book_title, user_table, book_table, W1, b1, W2, b2, W3, b3):
    ue, be = _sc_gather()(
        user_table,
        user_id.astype(jnp.int32),
        book_table,
        book_title.astype(jnp.int32),
    )

    w1u = W1[:EMB_DIM]
    w1b = W1[EMB_DIM:]
    b1r = b1.reshape(1, -1)
    b2r = b2.reshape(1, -1)
    w3r = W3.reshape(1, -1)
    b3r = b3.reshape(1, 1)

    grid = BATCH // MLP_BLOCK
    full = lambda i: (0, 0)
    out = pl.pallas_call(
        _mlp_body,
        grid=(grid,),
        in_specs=[
            pl.BlockSpec((MLP_BLOCK, EMB_DIM), lambda i: (i, 0)),
            pl.BlockSpec((MLP_BLOCK, EMB_DIM), lambda i: (i, 0)),
            pl.BlockSpec((EMB_DIM, 256), full),
            pl.BlockSpec((EMB_DIM, 256), full),
            pl.BlockSpec((1, 256), full),
            pl.BlockSpec((256, 64), full),
            pl.BlockSpec((1, 64), full),
            pl.BlockSpec((1, 64), full),
            pl.BlockSpec((1, 1), full),
        ],
        out_specs=pl.BlockSpec((MLP_BLOCK, 1), lambda i: (i, 0)),
        out_shape=jax.ShapeDtypeStruct((BATCH, 1), jnp.float32),
    )(ue, be, w1u, w1b, b1r, W2, b2r, w3r, b3r)
    return out


# speedup vs baseline: 1.6350x; 1.6350x over previous
import functools

import jax
import jax.numpy as jnp
from jax import lax
from jax.experimental import pallas as pl
from jax.experimental.pallas import tpu as pltpu
from jax.experimental.pallas import tpu_sc as plsc

EMB_DIM = 64
BATCH = 16384
NUM_WORKERS = 32
ROWS_PER_WORKER = BATCH // NUM_WORKERS
GATHER_CHUNK = 256
MLP_BLOCK = 2048


def _sc_gather_body(user_table, user_id, book_table, book_id,
                    ue_out, be_out,
                    uidx_v, urows_v, bidx_v, brows_v, usem, bsem):
    wid = lax.axis_index("s") * 2 + lax.axis_index("c")
    base = wid * ROWS_PER_WORKER
    pltpu.sync_copy(user_id.at[pl.ds(base, ROWS_PER_WORKER)], uidx_v)
    pltpu.sync_copy(book_id.at[pl.ds(base, ROWS_PER_WORKER)], bidx_v)

    for c in range(ROWS_PER_WORKER // GATHER_CHUNK):
        off = c * GATHER_CHUNK

        def step(j, carry):
            uvec = uidx_v[pl.ds(off + j * 16, 16)]
            bvec = bidx_v[pl.ds(off + j * 16, 16)]
            for k in range(16):
                pltpu.async_copy(user_table.at[pl.ds(uvec[k], 1), :],
                                 urows_v.at[pl.ds(j * 16 + k, 1), :], usem)
                pltpu.async_copy(book_table.at[pl.ds(bvec[k], 1), :],
                                 brows_v.at[pl.ds(j * 16 + k, 1), :], bsem)
            return carry

        lax.fori_loop(0, GATHER_CHUNK // 16, step, 0)
        pltpu.make_async_copy(
            user_table.at[pl.ds(0, GATHER_CHUNK), :], urows_v, usem).wait()
        pltpu.make_async_copy(
            book_table.at[pl.ds(0, GATHER_CHUNK), :], brows_v, bsem).wait()
        pltpu.sync_copy(urows_v, ue_out.at[pl.ds(base + off, GATHER_CHUNK)])
        pltpu.sync_copy(brows_v, be_out.at[pl.ds(base + off, GATHER_CHUNK)])


@functools.lru_cache(maxsize=1)
def _sc_gather():
    return pl.kernel(
        _sc_gather_body,
        out_type=[
            jax.ShapeDtypeStruct((BATCH, EMB_DIM), jnp.float32),
            jax.ShapeDtypeStruct((BATCH, EMB_DIM), jnp.float32),
        ],
        mesh=plsc.VectorSubcoreMesh(core_axis_name="c", subcore_axis_name="s"),
        scratch_types=[
            pltpu.VMEM((ROWS_PER_WORKER,), jnp.int32),
            pltpu.VMEM((GATHER_CHUNK, EMB_DIM), jnp.float32),
            pltpu.VMEM((ROWS_PER_WORKER,), jnp.int32),
            pltpu.VMEM((GATHER_CHUNK, EMB_DIM), jnp.float32),
            pltpu.SemaphoreType.DMA,
            pltpu.SemaphoreType.DMA,
        ],
    )


def _mlp_body(ue_ref, be_ref, w1u_ref, w1b_ref, b1_ref, w2_ref, b2_ref,
              w3_ref, b3_ref, out_ref):
    h = jnp.dot(ue_ref[...], w1u_ref[...], preferred_element_type=jnp.float32)
    h += jnp.dot(be_ref[...], w1b_ref[...], preferred_element_type=jnp.float32)
    h = jnp.maximum(h + b1_ref[...], 0.0)
    h = jnp.dot(h, w2_ref[...], preferred_element_type=jnp.float32)
    h = jnp.maximum(h + b2_ref[...], 0.0)
    out_ref[...] = (
        jnp.sum(h * w3_ref[...], axis=1, keepdims=True) + b3_ref[...]
    )


def kernel(user_id, book_title, user_table, book_table, W1, b1, W2, b2, W3, b3):
    ue, be = _sc_gather()(
        user_table,
        user_id.astype(jnp.int32),
        book_table,
        book_title.astype(jnp.int32),
    )

    w1u = W1[:EMB_DIM]
    w1b = W1[EMB_DIM:]
    b1r = b1.reshape(1, -1)
    b2r = b2.reshape(1, -1)
    w3r = W3.reshape(1, -1)
    b3r = b3.reshape(1, 1)

    grid = BATCH // MLP_BLOCK
    full = lambda i: (0, 0)
    out = pl.pallas_call(
        _mlp_body,
        grid=(grid,),
        in_specs=[
            pl.BlockSpec((MLP_BLOCK, EMB_DIM), lambda i: (i, 0)),
            pl.BlockSpec((MLP_BLOCK, EMB_DIM), lambda i: (i, 0)),
            pl.BlockSpec((EMB_DIM, 256), full),
            pl.BlockSpec((EMB_DIM, 256), full),
            pl.BlockSpec((1, 256), full),
            pl.BlockSpec((256, 64), full),
            pl.BlockSpec((1, 64), full),
            pl.BlockSpec((1, 64), full),
            pl.BlockSpec((1, 1), full),
        ],
        out_specs=pl.BlockSpec((MLP_BLOCK, 1), lambda i: (i, 0)),
        out_shape=jax.ShapeDtypeStruct((BATCH, 1), jnp.float32),
    )(ue, be, w1u, w1b, b1r, W2, b2r, w3r, b3r)
    return out


# speedup vs baseline: 2.0251x; 1.2386x over previous
import functools

import jax
import jax.numpy as jnp
from jax import lax
from jax.experimental import pallas as pl
from jax.experimental.pallas import tpu as pltpu
from jax.experimental.pallas import tpu_sc as plsc

EMB_DIM = 64
BATCH = 16384
NUM_WORKERS = 32
ROWS_PER_WORKER = BATCH // NUM_WORKERS
GATHER_CHUNK = 256
MLP_BLOCK = 2048
TR_BLOCK = 8192


def _transpose_body(in_ref, out_ref):
    out_ref[...] = in_ref[...].T


def _transpose(table_t):
    v = table_t.shape[1]
    grid = (v + TR_BLOCK - 1) // TR_BLOCK
    return pl.pallas_call(
        _transpose_body,
        grid=(grid,),
        in_specs=[pl.BlockSpec((EMB_DIM, TR_BLOCK), lambda i: (0, i))],
        out_specs=pl.BlockSpec((TR_BLOCK, EMB_DIM), lambda i: (i, 0)),
        out_shape=jax.ShapeDtypeStruct((v, EMB_DIM), jnp.float32),
    )(table_t)


def _sc_gather_body(user_table, user_id, book_table, book_id,
                    ue_out, be_out,
                    uidx_v, urows_v, bidx_v, brows_v, usem, bsem):
    wid = lax.axis_index("s") * 2 + lax.axis_index("c")
    base = wid * ROWS_PER_WORKER
    pltpu.sync_copy(user_id.at[pl.ds(base, ROWS_PER_WORKER)], uidx_v)
    pltpu.sync_copy(book_id.at[pl.ds(base, ROWS_PER_WORKER)], bidx_v)

    for c in range(ROWS_PER_WORKER // GATHER_CHUNK):
        off = c * GATHER_CHUNK

        def step(j, carry):
            uvec = uidx_v[pl.ds(off + j * 16, 16)]
            bvec = bidx_v[pl.ds(off + j * 16, 16)]
            for k in range(16):
                pltpu.async_copy(user_table.at[pl.ds(uvec[k], 1), :],
                                 urows_v.at[pl.ds(j * 16 + k, 1), :], usem)
                pltpu.async_copy(book_table.at[pl.ds(bvec[k], 1), :],
                                 brows_v.at[pl.ds(j * 16 + k, 1), :], bsem)
            return carry

        lax.fori_loop(0, GATHER_CHUNK // 16, step, 0)
        pltpu.make_async_copy(
            user_table.at[pl.ds(0, GATHER_CHUNK), :], urows_v, usem).wait()
        pltpu.make_async_copy(
            book_table.at[pl.ds(0, GATHER_CHUNK), :], brows_v, bsem).wait()
        pltpu.sync_copy(urows_v, ue_out.at[pl.ds(base + off, GATHER_CHUNK)])
        pltpu.sync_copy(brows_v, be_out.at[pl.ds(base + off, GATHER_CHUNK)])


@functools.lru_cache(maxsize=1)
def _sc_gather():
    return pl.kernel(
        _sc_gather_body,
        out_type=[
            jax.ShapeDtypeStruct((BATCH, EMB_DIM), jnp.float32),
            jax.ShapeDtypeStruct((BATCH, EMB_DIM), jnp.float32),
        ],
        mesh=plsc.VectorSubcoreMesh(core_axis_name="c", subcore_axis_name="s"),
        scratch_types=[
            pltpu.VMEM((ROWS_PER_WORKER,), jnp.int32),
            pltpu.VMEM((GATHER_CHUNK, EMB_DIM), jnp.float32),
            pltpu.VMEM((ROWS_PER_WORKER,), jnp.int32),
            pltpu.VMEM((GATHER_CHUNK, EMB_DIM), jnp.float32),
            pltpu.SemaphoreType.DMA,
            pltpu.SemaphoreType.DMA,
        ],
    )


def _mlp_body(ue_ref, be_ref, w1u_ref, w1b_ref, b1_ref, w2_ref, b2_ref,
              w3_ref, b3_ref, out_ref):
    h = jnp.dot(ue_ref[...], w1u_ref[...], preferred_element_type=jnp.float32)
    h += jnp.dot(be_ref[...], w1b_ref[...], preferred_element_type=jnp.float32)
    h = jnp.maximum(h + b1_ref[...], 0.0)
    h = jnp.dot(h, w2_ref[...], preferred_element_type=jnp.float32)
    h = jnp.maximum(h + b2_ref[...], 0.0)
    out_ref[...] = (
        jnp.sum(h * w3_ref[...], axis=1, keepdims=True) + b3_ref[...]
    )


def kernel(user_id, book_title, user_table, book_table, W1, b1, W2, b2, W3, b3):
    ut = _transpose(user_table.T)
    bt = _transpose(book_table.T)
    ue, be = _sc_gather()(
        ut,
        user_id.astype(jnp.int32),
        bt,
        book_title.astype(jnp.int32),
    )

    w1u = W1[:EMB_DIM]
    w1b = W1[EMB_DIM:]
    b1r = b1.reshape(1, -1)
    b2r = b2.reshape(1, -1)
    w3r = W3.reshape(1, -1)
    b3r = b3.reshape(1, 1)

    grid = BATCH // MLP_BLOCK
    full = lambda i: (0, 0)
    out = pl.pallas_call(
        _mlp_body,
        grid=(grid,),
        in_specs=[
            pl.BlockSpec((MLP_BLOCK, EMB_DIM), lambda i: (i, 0)),
            pl.BlockSpec((MLP_BLOCK, EMB_DIM), lambda i: (i, 0)),
            pl.BlockSpec((EMB_DIM, 256), full),
            pl.BlockSpec((EMB_DIM, 256), full),
            pl.BlockSpec((1, 256), full),
            pl.BlockSpec((256, 64), full),
            pl.BlockSpec((1, 64), full),
            pl.BlockSpec((1, 64), full),
            pl.BlockSpec((1, 1), full),
        ],
        out_specs=pl.BlockSpec((MLP_BLOCK, 1), lambda i: (i, 0)),
        out_shape=jax.ShapeDtypeStruct((BATCH, 1), jnp.float32),
    )(ue, be, w1u, w1b, b1r, W2, b2r, w3r, b3r)
    return out


# speedup vs baseline: 2.0274x; 1.0011x over previous
import functools

import jax
import jax.numpy as jnp
from jax import lax
from jax.experimental import pallas as pl
from jax.experimental.pallas import tpu as pltpu
from jax.experimental.pallas import tpu_sc as plsc

EMB_DIM = 64
BATCH = 16384
NUM_WORKERS = 32
ROWS_PER_WORKER = BATCH // NUM_WORKERS
GATHER_CHUNK = 256
MLP_BLOCK = 2048
TR_BLOCK = 8192


def _transpose_body(in_ref, out_ref):
    x = in_ref[...]
    half = TR_BLOCK // 2
    out_ref[...] = jnp.concatenate(
        [x[:, :half].T, x[:, half:].T], axis=1)


def _transpose(table_t):
    v = table_t.shape[1]
    grid = (v + TR_BLOCK - 1) // TR_BLOCK
    return pl.pallas_call(
        _transpose_body,
        grid=(grid,),
        in_specs=[pl.BlockSpec((EMB_DIM, TR_BLOCK), lambda i: (0, i))],
        out_specs=pl.BlockSpec((TR_BLOCK // 2, 2 * EMB_DIM), lambda i: (i, 0)),
        out_shape=jax.ShapeDtypeStruct(
            (grid * (TR_BLOCK // 2), 2 * EMB_DIM), jnp.float32),
    )(table_t)


def _sc_gather_body(user_table, user_id, book_table, book_id,
                    ue_out, be_out,
                    uidx_v, urows_v, bidx_v, brows_v, usem, bsem):
    wid = lax.axis_index("s") * 2 + lax.axis_index("c")
    base = wid * ROWS_PER_WORKER
    pltpu.sync_copy(user_id.at[pl.ds(base, ROWS_PER_WORKER)], uidx_v)
    pltpu.sync_copy(book_id.at[pl.ds(base, ROWS_PER_WORKER)], bidx_v)

    for c in range(ROWS_PER_WORKER // GATHER_CHUNK):
        off = c * GATHER_CHUNK

        def step(j, carry):
            uvec = uidx_v[pl.ds(off + j * 16, 16)]
            bvec = bidx_v[pl.ds(off + j * 16, 16)]
            uo = ((uvec >> 13) << 19) + ((uvec & 4095) << 7) + \
                (((uvec >> 12) & 1) << 6)
            bo = ((bvec >> 13) << 19) + ((bvec & 4095) << 7) + \
                (((bvec >> 12) & 1) << 6)
            for k in range(16):
                s = (j * 16 + k) * EMB_DIM
                uok = pl.multiple_of(uo[k], EMB_DIM)
                bok = pl.multiple_of(bo[k], EMB_DIM)
                pltpu.async_copy(user_table.at[pl.ds(uok, EMB_DIM)],
                                 urows_v.at[pl.ds(s, EMB_DIM)], usem)
                pltpu.async_copy(book_table.at[pl.ds(bok, EMB_DIM)],
                                 brows_v.at[pl.ds(s, EMB_DIM)], bsem)
            return carry

        lax.fori_loop(0, GATHER_CHUNK // 16, step, 0)
        pltpu.make_async_copy(
            user_table.at[pl.ds(0, GATHER_CHUNK * EMB_DIM)],
            urows_v, usem).wait()
        pltpu.make_async_copy(
            book_table.at[pl.ds(0, GATHER_CHUNK * EMB_DIM)],
            brows_v, bsem).wait()
        pltpu.sync_copy(
            urows_v,
            ue_out.at[pl.ds((base + off) * EMB_DIM, GATHER_CHUNK * EMB_DIM)])
        pltpu.sync_copy(
            brows_v,
            be_out.at[pl.ds((base + off) * EMB_DIM, GATHER_CHUNK * EMB_DIM)])


@functools.lru_cache(maxsize=1)
def _sc_gather():
    return pl.kernel(
        _sc_gather_body,
        out_type=[
            jax.ShapeDtypeStruct((BATCH * EMB_DIM,), jnp.float32),
            jax.ShapeDtypeStruct((BATCH * EMB_DIM,), jnp.float32),
        ],
        mesh=plsc.VectorSubcoreMesh(core_axis_name="c", subcore_axis_name="s"),
        scratch_types=[
            pltpu.VMEM((ROWS_PER_WORKER,), jnp.int32),
            pltpu.VMEM((GATHER_CHUNK * EMB_DIM,), jnp.float32),
            pltpu.VMEM((ROWS_PER_WORKER,), jnp.int32),
            pltpu.VMEM((GATHER_CHUNK * EMB_DIM,), jnp.float32),
            pltpu.SemaphoreType.DMA,
            pltpu.SemaphoreType.DMA,
        ],
    )


def _mlp_body(ue_ref, be_ref, w1u_ref, w1b_ref, b1_ref, w2_ref, b2_ref,
              w3_ref, b3_ref, out_ref):
    h = jnp.dot(ue_ref[...], w1u_ref[...], preferred_element_type=jnp.float32)
    h += jnp.dot(be_ref[...], w1b_ref[...], preferred_element_type=jnp.float32)
    h = jnp.maximum(h + b1_ref[...], 0.0)
    h = jnp.dot(h, w2_ref[...], preferred_element_type=jnp.float32)
    h = jnp.maximum(h + b2_ref[...], 0.0)
    out_ref[...] = (
        jnp.sum(h * w3_ref[...], axis=1, keepdims=True) + b3_ref[...]
    )


def kernel(user_id, book_title, user_table, book_table, W1, b1, W2, b2, W3, b3):
    ut = _transpose(user_table.T)
    bt = _transpose(book_table.T)
    ue, be = _sc_gather()(
        ut.reshape(-1),
        user_id.astype(jnp.int32),
        bt.reshape(-1),
        book_title.astype(jnp.int32),
    )
    ue = ue.reshape(BATCH, EMB_DIM)
    be = be.reshape(BATCH, EMB_DIM)

    w1u = W1[:EMB_DIM]
    w1b = W1[EMB_DIM:]
    b1r = b1.reshape(1, -1)
    b2r = b2.reshape(1, -1)
    w3r = W3.reshape(1, -1)
    b3r = b3.reshape(1, 1)

    grid = BATCH // MLP_BLOCK
    full = lambda i: (0, 0)
    out = pl.pallas_call(
        _mlp_body,
        grid=(grid,),
        in_specs=[
            pl.BlockSpec((MLP_BLOCK, EMB_DIM), lambda i: (i, 0)),
            pl.BlockSpec((MLP_BLOCK, EMB_DIM), lambda i: (i, 0)),
            pl.BlockSpec((EMB_DIM, 256), full),
            pl.BlockSpec((EMB_DIM, 256), full),
            pl.BlockSpec((1, 256), full),
            pl.BlockSpec((256, 64), full),
            pl.BlockSpec((1, 64), full),
            pl.BlockSpec((1, 64), full),
            pl.BlockSpec((1, 1), full),
        ],
        out_specs=pl.BlockSpec((MLP_BLOCK, 1), lambda i: (i, 0)),
        out_shape=jax.ShapeDtypeStruct((BATCH, 1), jnp.float32),
    )(ue, be, w1u, w1b, b1r, W2, b2r, w3r, b3r)
    return out


# speedup vs baseline: 2.0500x; 1.0111x over previous
import functools

import jax
import jax.numpy as jnp
from jax import lax
from jax.experimental import pallas as pl
from jax.experimental.pallas import tpu as pltpu
from jax.experimental.pallas import tpu_sc as plsc

EMB_DIM = 64
BATCH = 16384
NUM_WORKERS = 32
ROWS_PER_WORKER = BATCH // NUM_WORKERS
GATHER_CHUNK = 256
MLP_BLOCK = 2048
TR_BLOCK = 8192
WORDS = EMB_DIM // 2


def _round_bf16_bits(x):
    u = lax.bitcast_convert_type(x, jnp.uint32)
    return (u + 0x7FFF + ((u >> 16) & 1)) >> 16


def _transpose_body(in_ref, out_ref):
    x = in_ref[...]
    lo = _round_bf16_bits(x[:WORDS, :])
    hi = _round_bf16_bits(x[WORDS:, :])
    packed = lax.bitcast_convert_type(
        lo | (hi << 16), jnp.int32)
    q = TR_BLOCK // 4
    parts = [packed[:, i * q:(i + 1) * q].T for i in range(4)]
    out_ref[...] = jnp.concatenate(parts, axis=1)


def _transpose(table_t):
    v = table_t.shape[1]
    grid = (v + TR_BLOCK - 1) // TR_BLOCK
    return pl.pallas_call(
        _transpose_body,
        grid=(grid,),
        in_specs=[pl.BlockSpec((EMB_DIM, TR_BLOCK), lambda i: (0, i))],
        out_specs=pl.BlockSpec((TR_BLOCK // 4, 2 * EMB_DIM), lambda i: (i, 0)),
        out_shape=jax.ShapeDtypeStruct(
            (grid * (TR_BLOCK // 4), 2 * EMB_DIM), jnp.int32),
    )(table_t)


def _sc_gather_body(user_table, user_id, book_table, book_id,
                    ue_out, be_out,
                    uidx_v, urows_v, bidx_v, brows_v, usem, bsem):
    wid = lax.axis_index("s") * 2 + lax.axis_index("c")
    base = wid * ROWS_PER_WORKER
    pltpu.sync_copy(user_id.at[pl.ds(base, ROWS_PER_WORKER)], uidx_v)
    pltpu.sync_copy(book_id.at[pl.ds(base, ROWS_PER_WORKER)], bidx_v)

    for c in range(ROWS_PER_WORKER // GATHER_CHUNK):
        off = c * GATHER_CHUNK

        def step(j, carry):
            uvec = uidx_v[pl.ds(off + j * 16, 16)]
            bvec = bidx_v[pl.ds(off + j * 16, 16)]
            uo = ((uvec >> 13) << 18) + ((uvec & 2047) << 7) + \
                (((uvec >> 11) & 3) << 5)
            bo = ((bvec >> 13) << 18) + ((bvec & 2047) << 7) + \
                (((bvec >> 11) & 3) << 5)
            for k in range(16):
                s = (j * 16 + k) * WORDS
                uok = pl.multiple_of(uo[k], WORDS)
                bok = pl.multiple_of(bo[k], WORDS)
                pltpu.async_copy(user_table.at[pl.ds(uok, WORDS)],
                                 urows_v.at[pl.ds(s, WORDS)], usem)
                pltpu.async_copy(book_table.at[pl.ds(bok, WORDS)],
                                 brows_v.at[pl.ds(s, WORDS)], bsem)
            return carry

        lax.fori_loop(0, GATHER_CHUNK // 16, step, 0)
        pltpu.make_async_copy(
            user_table.at[pl.ds(0, GATHER_CHUNK * WORDS)],
            urows_v, usem).wait()
        pltpu.make_async_copy(
            book_table.at[pl.ds(0, GATHER_CHUNK * WORDS)],
            brows_v, bsem).wait()
        pltpu.sync_copy(
            urows_v,
            ue_out.at[pl.ds((base + off) * WORDS, GATHER_CHUNK * WORDS)])
        pltpu.sync_copy(
            brows_v,
            be_out.at[pl.ds((base + off) * WORDS, GATHER_CHUNK * WORDS)])


@functools.lru_cache(maxsize=1)
def _sc_gather():
    return pl.kernel(
        _sc_gather_body,
        out_type=[
            jax.ShapeDtypeStruct((BATCH * WORDS,), jnp.int32),
            jax.ShapeDtypeStruct((BATCH * WORDS,), jnp.int32),
        ],
        mesh=plsc.VectorSubcoreMesh(core_axis_name="c", subcore_axis_name="s"),
        scratch_types=[
            pltpu.VMEM((ROWS_PER_WORKER,), jnp.int32),
            pltpu.VMEM((GATHER_CHUNK * WORDS,), jnp.int32),
            pltpu.VMEM((ROWS_PER_WORKER,), jnp.int32),
            pltpu.VMEM((GATHER_CHUNK * WORDS,), jnp.int32),
            pltpu.SemaphoreType.DMA,
            pltpu.SemaphoreType.DMA,
        ],
    )


def _unpack_bf16_pair(words):
    lo = lax.bitcast_convert_type(words << 16, jnp.float32)
    hi = lax.bitcast_convert_type(words & jnp.int32(-65536), jnp.float32)
    return lo, hi


def _mlp_body(ue_ref, be_ref, w1ul_ref, w1uh_ref, w1bl_ref, w1bh_ref,
              b1_ref, w2_ref, b2_ref, w3_ref, b3_ref, out_ref):
    ue_lo, ue_hi = _unpack_bf16_pair(ue_ref[...])
    be_lo, be_hi = _unpack_bf16_pair(be_ref[...])
    h = jnp.dot(ue_lo, w1ul_ref[...], preferred_element_type=jnp.float32)
    h += jnp.dot(ue_hi, w1uh_ref[...], preferred_element_type=jnp.float32)
    h += jnp.dot(be_lo, w1bl_ref[...], preferred_element_type=jnp.float32)
    h += jnp.dot(be_hi, w1bh_ref[...], preferred_element_type=jnp.float32)
    h = jnp.maximum(h + b1_ref[...], 0.0)
    h = jnp.dot(h, w2_ref[...], preferred_element_type=jnp.float32)
    h = jnp.maximum(h + b2_ref[...], 0.0)
    out_ref[...] = (
        jnp.sum(h * w3_ref[...], axis=1, keepdims=True) + b3_ref[...]
    )


def kernel(user_id, book_title, user_table, book_table, W1, b1, W2, b2, W3, b3):
    ut = _transpose(user_table.T)
    bt = _transpose(book_table.T)
    ue, be = _sc_gather()(
        ut.reshape(-1),
        user_id.astype(jnp.int32),
        bt.reshape(-1),
        book_title.astype(jnp.int32),
    )
    ue = ue.reshape(BATCH, WORDS)
    be = be.reshape(BATCH, WORDS)

    w1ul = W1[:WORDS]
    w1uh = W1[WORDS:EMB_DIM]
    w1bl = W1[EMB_DIM:EMB_DIM + WORDS]
    w1bh = W1[EMB_DIM + WORDS:]
    b1r = b1.reshape(1, -1)
    b2r = b2.reshape(1, -1)
    w3r = W3.reshape(1, -1)
    b3r = b3.reshape(1, 1)

    grid = BATCH // MLP_BLOCK
    full = lambda i: (0, 0)
    out = pl.pallas_call(
        _mlp_body,
        grid=(grid,),
        in_specs=[
            pl.BlockSpec((MLP_BLOCK, WORDS), lambda i: (i, 0)),
            pl.BlockSpec((MLP_BLOCK, WORDS), lambda i: (i, 0)),
            pl.BlockSpec((WORDS, 256), full),
            pl.BlockSpec((WORDS, 256), full),
            pl.BlockSpec((WORDS, 256), full),
            pl.BlockSpec((WORDS, 256), full),
            pl.BlockSpec((1, 256), full),
            pl.BlockSpec((256, 64), full),
            pl.BlockSpec((1, 64), full),
            pl.BlockSpec((1, 64), full),
            pl.BlockSpec((1, 1), full),
        ],
        out_specs=pl.BlockSpec((MLP_BLOCK, 1), lambda i: (i, 0)),
        out_shape=jax.ShapeDtypeStruct((BATCH, 1), jnp.float32),
    )(ue, be, w1ul, w1uh, w1bl, w1bh, b1r, W2, b2r, w3r, b3r)
    return out
